# Initial kernel scaffold; baseline (speedup 1.0000x reference)
import jax
import jax.numpy as jnp
from jax.experimental import pallas as pl


def kernel(node_emb_inds, edge_src, edge_dst, edge_type, node_emb, W_i, W_o, W_c, W_f, U_i, U_o, U_c, U_f, b_i, b_o, b_c, b_f):
    raise NotImplementedError("write your pallas kernel here")


# speedup vs baseline: 2.9218x; 2.9218x over previous
import functools

import jax
import jax.numpy as jnp
from jax import lax
from jax.experimental import pallas as pl
from jax.experimental.pallas import tpu as pltpu
from jax.experimental.pallas import tpu_sc as plsc

NODE_CT = 10000
EMB = 128
S = 128
NT = 17
L = 4
EPL = 40000

NP = 10240
EB = 512
PADDED = 49152
NBLK = PADDED // EB
RPT = NP // 16
ECH = 128
GCH = PADDED // 32 // ECH
SCH = PADDED // 16 // ECH

_mesh = plsc.VectorSubcoreMesh(core_axis_name="c", subcore_axis_name="s")


def _sc_states_gather(inds3d, emb2):
    @functools.partial(
        pl.kernel, mesh=_mesh,
        out_type=jax.ShapeDtypeStruct((NP, EMB), jnp.float32),
        scratch_types=[
            pltpu.VMEM((5, 64), jnp.int32),
            pltpu.VMEM((64, EMB), jnp.float32),
            pltpu.SemaphoreType.DMA,
        ],
    )
    def k(inds_hbm, emb_hbm, out_hbm, idx_v, buf_v, sem):
        wid = lax.axis_index("s") * 2 + lax.axis_index("c")
        pltpu.sync_copy(inds_hbm.at[wid], idx_v)
        for j in range(5):
            pltpu.async_copy(emb_hbm.at[idx_v.at[j]], buf_v, sem).wait()
            pltpu.sync_copy(buf_v, out_hbm.at[pl.ds((wid * 5 + j) * 64, 64)])

    return k(inds3d, emb2)


def _sc_edge_gather(srcp3d, dstp3d, hc, wfp):
    @functools.partial(
        pl.kernel, mesh=_mesh,
        out_type=(jax.ShapeDtypeStruct((PADDED, 2 * S), jnp.float32),
                  jax.ShapeDtypeStruct((PADDED, S), jnp.float32)),
        scratch_types=[
            pltpu.VMEM((GCH, ECH), jnp.int32),
            pltpu.VMEM((GCH, ECH), jnp.int32),
            pltpu.VMEM((ECH, 2 * S), jnp.float32),
            pltpu.VMEM((ECH, S), jnp.float32),
            pltpu.SemaphoreType.DMA,
            pltpu.SemaphoreType.DMA,
        ],
    )
    def k(src_hbm, dst_hbm, hc_hbm, wfp_hbm, hcs_hbm, wfd_hbm,
          idxs_v, idxd_v, hcbuf, wfbuf, sem1, sem2):
        wid = lax.axis_index("s") * 2 + lax.axis_index("c")
        base = wid * GCH
        pltpu.sync_copy(src_hbm.at[wid], idxs_v)
        pltpu.sync_copy(dst_hbm.at[wid], idxd_v)
        for j in range(GCH):
            a = pltpu.async_copy(hc_hbm.at[idxs_v.at[j]], hcbuf, sem1)
            b = pltpu.async_copy(wfp_hbm.at[idxd_v.at[j]], wfbuf, sem2)
            a.wait()
            b.wait()
            row0 = (base + j) * ECH
            pltpu.sync_copy(hcbuf, hcs_hbm.at[pl.ds(row0, ECH)])
            pltpu.sync_copy(wfbuf, wfd_hbm.at[pl.ds(row0, ECH)])

    return k(srcp3d, dstp3d, hc, wfp)


def _sc_dmask(dst3d, ones128, zrows):
    @functools.partial(
        pl.kernel, mesh=_mesh,
        out_type=jax.ShapeDtypeStruct((L, NP, S), jnp.float32),
        scratch_types=[
            pltpu.VMEM((20, ECH), jnp.int32),
            pltpu.VMEM((ECH, S), jnp.float32),
            pltpu.VMEM((64, S), jnp.float32),
            pltpu.VMEM_SHARED((NP, S), jnp.float32),
            pltpu.SemaphoreType.DMA,
        ],
    )
    def k(dst_hbm, ones_hbm, z_hbm, out_hbm, idx_v, ones_v, obuf, smem, sem):
        cid = lax.axis_index("c")
        sid = lax.axis_index("s")
        pltpu.sync_copy(ones_hbm, ones_v)
        for ll in range(2):
            l = 2 * cid + ll
            pltpu.sync_copy(dst_hbm.at[l * 16 + sid], idx_v)
            pltpu.sync_copy(z_hbm, obuf)
            for t in range(RPT // 64):
                pltpu.sync_copy(obuf, smem.at[pl.ds(sid * RPT + t * 64, 64)])
            plsc.subcore_barrier()
            for j in range(20):
                pltpu.sync_copy(ones_v, smem.at[idx_v.at[j]], add=True)
            plsc.subcore_barrier()
            for t in range(RPT // 64):
                r0 = sid * RPT + t * 64
                pltpu.sync_copy(smem.at[pl.ds(r0, 64)], obuf)
                pltpu.sync_copy(obuf, out_hbm.at[l, pl.ds(r0, 64)])
            plsc.subcore_barrier()

    return k(dst3d, ones128, zrows)


def _sc_segsum(dstp3d, msg, zrows):
    @functools.partial(
        pl.kernel, mesh=_mesh,
        out_type=jax.ShapeDtypeStruct((4, NP, S), jnp.float32),
        scratch_types=[
            pltpu.VMEM((SCH, ECH), jnp.int32),
            pltpu.VMEM((ECH, S), jnp.float32),
            pltpu.VMEM((64, S), jnp.float32),
            pltpu.VMEM_SHARED((NP, S), jnp.float32),
            pltpu.SemaphoreType.DMA,
        ],
    )
    def k(dst_hbm, m_hbm, z_hbm, out_hbm, idx_v, mbuf, obuf, smem, sem):
        cid = lax.axis_index("c")
        sid = lax.axis_index("s")
        pltpu.sync_copy(dst_hbm.at[sid], idx_v)
        for gg in range(2):
            g = 2 * cid + gg
            pltpu.sync_copy(z_hbm, obuf)
            for t in range(RPT // 64):
                pltpu.sync_copy(obuf, smem.at[pl.ds(sid * RPT + t * 64, 64)])
            plsc.subcore_barrier()
            for j in range(SCH):
                row0 = sid * SCH * ECH + j * ECH
                pltpu.sync_copy(m_hbm.at[g, pl.ds(row0, ECH)], mbuf)
                pltpu.sync_copy(mbuf, smem.at[idx_v.at[j]], add=True)
            plsc.subcore_barrier()
            for t in range(RPT // 64):
                r0 = sid * RPT + t * 64
                pltpu.sync_copy(smem.at[pl.ds(r0, 64)], obuf)
                pltpu.sync_copy(obuf, out_hbm.at[g, pl.ds(r0, 64)])
            plsc.subcore_barrier()

    return k(dstp3d, msg, zrows)


def _tc_wx(states, wall):
    RB = 1024

    def body(s_ref, w_ref, wx_ref, wf_ref):
        m = jnp.dot(s_ref[...], w_ref[...], preferred_element_type=jnp.float32)
        wx_ref[...] = m[:, :3 * S]
        wf_ref[...] = m[:, 3 * S:]

    return pl.pallas_call(
        body,
        grid=(NP // RB,),
        in_specs=[
            pl.BlockSpec((RB, EMB), lambda b: (b, 0)),
            pl.BlockSpec((EMB, 4 * S), lambda b: (0, 0)),
        ],
        out_specs=(pl.BlockSpec((RB, 3 * S), lambda b: (b, 0)),
                   pl.BlockSpec((RB, S), lambda b: (b, 0))),
        out_shape=(jax.ShapeDtypeStruct((NP, 3 * S), jnp.float32),
                   jax.ShapeDtypeStruct((NP, S), jnp.float32)),
    )(states, wall)


def _tc_messages(blk_t, hcsrc, wfdst, ucat, bfrow):
    def body(bt_ref, hc_ref, wf_ref, u_ref, bf_ref, m_ref):
        del bt_ref
        m = jnp.dot(hc_ref[:, :S], u_ref[0],
                    preferred_element_type=jnp.float32)
        m_ref[0] = m[:, 0:S]
        m_ref[1] = m[:, S:2 * S]
        m_ref[2] = m[:, 2 * S:3 * S]
        f = jax.nn.sigmoid(wf_ref[...] + m[:, 3 * S:] + bf_ref[0:1, :])
        m_ref[3] = f * hc_ref[:, S:]

    grid_spec = pltpu.PrefetchScalarGridSpec(
        num_scalar_prefetch=1,
        grid=(NBLK,),
        in_specs=[
            pl.BlockSpec((EB, 2 * S), lambda b, bt: (b, 0)),
            pl.BlockSpec((EB, S), lambda b, bt: (b, 0)),
            pl.BlockSpec((1, S, 4 * S), lambda b, bt: (bt[b], 0, 0)),
            pl.BlockSpec((8, S), lambda b, bt: (0, 0)),
        ],
        out_specs=pl.BlockSpec((4, EB, S), lambda b, bt: (0, b, 0)),
    )
    out_shape = jax.ShapeDtypeStruct((4, PADDED, S), jnp.float32)
    return pl.pallas_call(body, grid_spec=grid_spec, out_shape=out_shape)(
        blk_t, hcsrc, wfdst, ucat, bfrow)


def _gate_math(wx, segi, sego, segc, dm, b3):
    i_g = jax.nn.sigmoid(wx[:, 0:S] * dm + segi + b3[0:1, 0:S])
    o_g = jax.nn.sigmoid(wx[:, S:2 * S] * dm + sego + b3[0:1, S:2 * S])
    ch = jnp.tanh(wx[:, 2 * S:] * dm + segc + b3[0:1, 2 * S:])
    return i_g, o_g, ch


def _tc_gates0(wx, dmc, b3):
    RB = 1280

    def body(wx_ref, dm_ref, b3_ref, hc_ref):
        dm = (dm_ref[:, 0:1] > 0).astype(jnp.float32)
        i_g, o_g, ch = _gate_math(wx_ref[...], 0.0, 0.0, 0.0, dm, b3_ref[...])
        c = (i_g * ch) * dm
        h = (o_g * jnp.tanh(c)) * dm
        hc_ref[:, :S] = h
        hc_ref[:, S:] = c

    return pl.pallas_call(
        body,
        grid=(NP // RB,),
        in_specs=[
            pl.BlockSpec((RB, 3 * S), lambda b: (b, 0)),
            pl.BlockSpec((RB, S), lambda b: (b, 0)),
            pl.BlockSpec((8, 3 * S), lambda b: (0, 0)),
        ],
        out_specs=pl.BlockSpec((RB, 2 * S), lambda b: (b, 0)),
        out_shape=jax.ShapeDtypeStruct((NP, 2 * S), jnp.float32),
    )(wx, dmc, b3)


def _tc_gates(wx, segi, sego, segc, segfc, dmc, hc, b3):
    RB = 1280

    def body(wx_ref, si, so, sc_, sfc, dm_ref, hc_in, b3_ref, hc_ref):
        dm = (dm_ref[:, 0:1] > 0).astype(jnp.float32)
        i_g, o_g, ch = _gate_math(wx_ref[...], si[...], so[...], sc_[...],
                                  dm, b3_ref[...])
        par = i_g * ch + sfc[...]
        c = hc_in[:, S:] + par * dm
        h = hc_in[:, :S] + (o_g * jnp.tanh(c)) * dm
        hc_ref[:, :S] = h
        hc_ref[:, S:] = c

    return pl.pallas_call(
        body,
        grid=(NP // RB,),
        in_specs=[
            pl.BlockSpec((RB, 3 * S), lambda b: (b, 0)),
            pl.BlockSpec((RB, S), lambda b: (b, 0)),
            pl.BlockSpec((RB, S), lambda b: (b, 0)),
            pl.BlockSpec((RB, S), lambda b: (b, 0)),
            pl.BlockSpec((RB, S), lambda b: (b, 0)),
            pl.BlockSpec((RB, S), lambda b: (b, 0)),
            pl.BlockSpec((RB, 2 * S), lambda b: (b, 0)),
            pl.BlockSpec((8, 3 * S), lambda b: (0, 0)),
        ],
        out_specs=pl.BlockSpec((RB, 2 * S), lambda b: (b, 0)),
        out_shape=jax.ShapeDtypeStruct((NP, 2 * S), jnp.float32),
        input_output_aliases={6: 0},
    )(wx, segi, sego, segc, segfc, dmc, hc, b3)


def _sorted_padded_edges(src, dst, et):
    order = jnp.argsort(et)
    counts = jnp.bincount(et, length=NT)
    pc = ((counts + EB - 1) // EB) * EB
    poff = jnp.concatenate([jnp.zeros((1,), pc.dtype), jnp.cumsum(pc)])[:NT]
    coff = jnp.concatenate([jnp.zeros((1,), counts.dtype),
                            jnp.cumsum(counts)])[:NT]
    src_s = src[order]
    dst_s = dst[order]
    p = jnp.arange(PADDED, dtype=jnp.int32)
    t = jnp.clip(jnp.searchsorted(poff, p, side="right") - 1, 0, NT - 1)
    r = p - poff[t]
    valid = r < counts[t]
    idx = jnp.clip(coff[t] + r, 0, EPL - 1)
    src_p = jnp.where(valid, src_s[idx], NODE_CT).astype(jnp.int32)
    dst_p = jnp.where(valid, dst_s[idx], NODE_CT).astype(jnp.int32)
    blk_t = jnp.clip(
        jnp.searchsorted(poff, jnp.arange(NBLK, dtype=jnp.int32) * EB,
                         side="right") - 1, 0, NT - 1).astype(jnp.int32)
    return (src_p.reshape(32, GCH, ECH), dst_p.reshape(32, GCH, ECH),
            dst_p.reshape(16, SCH, ECH), blk_t)


def kernel(node_emb_inds, edge_src, edge_dst, edge_type, node_emb,
           W_i, W_o, W_c, W_f, U_i, U_o, U_c, U_f, b_i, b_o, b_c, b_f):
    f32 = jnp.float32
    emb2 = jnp.concatenate([node_emb, jnp.zeros((1, EMB), f32)], axis=0)
    inds_p = jnp.concatenate([
        node_emb_inds.astype(jnp.int32),
        jnp.full((NP - NODE_CT,), NODE_CT, jnp.int32)]).reshape(32, 5, 64)
    wall = jnp.concatenate([W_i, W_o, W_c, W_f], axis=1)
    ucat = jnp.concatenate([U_i, U_o, U_c, U_f], axis=2)
    b3 = jnp.concatenate([jnp.full((8, S), b_i, f32),
                          jnp.full((8, S), b_o, f32),
                          jnp.full((8, S), b_c, f32)], axis=1)
    bfrow = jnp.full((8, S), b_f, f32)

    states = _sc_states_gather(inds_p, emb2)
    wx, wfp = _tc_wx(states, wall)

    dst_pad = jnp.concatenate(
        [edge_dst.astype(jnp.int32),
         jnp.full((L, 960), NODE_CT, jnp.int32)], axis=1)
    dst3d = dst_pad.reshape(L * 16, 20, ECH)
    ones128 = jnp.ones((ECH, S), f32)
    zrows = jnp.zeros((64, S), f32)
    dmask_all = _sc_dmask(dst3d, ones128, zrows)

    hc = _tc_gates0(wx, dmask_all[0], b3)

    preps = [_sorted_padded_edges(
        edge_src[l].astype(jnp.int32), edge_dst[l].astype(jnp.int32),
        edge_type[l].astype(jnp.int32)) for l in range(1, L)]
    xs = (jnp.stack([p[0] for p in preps]),
          jnp.stack([p[1] for p in preps]),
          jnp.stack([p[2] for p in preps]),
          jnp.stack([p[3] for p in preps]),
          dmask_all[1:L])

    def layer_body(hc_carry, x):
        srcp3d, dstp3d, dstp3ds, blk_t, dmc = x
        hcsrc, wfdst = _sc_edge_gather(srcp3d, dstp3d, hc_carry, wfp)
        msg = _tc_messages(blk_t, hcsrc, wfdst, ucat, bfrow)
        seg = _sc_segsum(dstp3ds, msg, zrows)
        hc_new = _tc_gates(wx, seg[0], seg[1], seg[2], seg[3],
                           dmc, hc_carry, b3)
        return hc_new, None

    hc, _ = lax.scan(layer_body, hc, xs)

    return hc[:NODE_CT, :S]
